# baseline (speedup 1.0000x reference)
import jax
import jax.numpy as jnp
from jax import lax
from jax.experimental import pallas as pl
from jax.experimental.pallas import tpu as pltpu
from jax.experimental.pallas import tpu_sc as plsc

_VOCAB = 32000
_D = 512
_T = 8192
_NC = 2
_NS = 16
_NW = _NC * _NS
_TPW = _T // _NW
_CHUNK = 64
_NCHUNK = _TPW // _CHUNK
_LANES = 16
_NVEC = _D // _LANES
_UNROLL = 4
_NBUF = 3


def _gat(vec, idx):
    dnums = lax.GatherDimensionNumbers(
        offset_dims=(), collapsed_slice_dims=(0,), start_index_map=(0,))
    return lax.gather(vec, idx[:, None], dnums, (1,),
                      mode=lax.GatherScatterMode.PROMISE_IN_BOUNDS)


def _bcast(vec, lane):
    return _gat(vec, jnp.full((_LANES,), lane, jnp.int32))


def _body(ids_hbm, cu_hbm, table_hbm, out_hbm, idx_v, rows_v, w_v, cu_v,
          acc_v, sems):
    cid = lax.axis_index("c")
    sid = lax.axis_index("s")
    wid = cid * _NS + sid
    base = wid * _TPW

    cu_d = pltpu.make_async_copy(cu_hbm, cu_v.at[pl.ds(0, 9)], sems.at[1])
    idx_d = pltpu.make_async_copy(
        ids_hbm.at[pl.ds(base, _TPW)], idx_v, sems.at[1])
    cu_d.start()
    idx_d.start()

    def gather_descr(c):
        slot = lax.rem(c, _NBUF) if not isinstance(c, int) else c % _NBUF
        return pltpu.make_async_copy(
            table_hbm.at[idx_v.at[pl.ds(c * _CHUNK, _CHUNK)]],
            rows_v.at[slot], sems.at[slot])

    idx_d.wait()
    cu_d.wait()
    gather_descr(0).start()
    gather_descr(1).start()

    lanes = lax.iota(jnp.int32, 16)
    cuv = cu_v[...]
    cks = [_bcast(cuv, k) for k in range(1, 9)]

    def wgrp(jj, carry):
        t = base + jj * _LANES + lanes
        seg = jnp.zeros((_LANES,), jnp.int32)
        for ck in cks:
            seg = seg + jnp.where(ck <= t, 1, 0)
        start = _gat(cuv, seg)
        end = _gat(cuv, seg + 1)
        pos = t - start
        seg_len = end - start
        w = ((pos + 1) * (seg_len - pos)).astype(jnp.float32)
        w_v[pl.ds(jj * _LANES, _LANES)] = w
        return carry

    lax.fori_loop(0, _TPW // _LANES, wgrp, jnp.int32(0))

    def chunk_body(c, accs):
        nxt = jnp.minimum(c + 2, _NCHUNK - 1)

        @pl.when(c + 2 < _NCHUNK)
        def _():
            gather_descr(nxt).start()

        gather_descr(c).wait()
        buf = lax.rem(c, _NBUF)

        def rowit(it, accs):
            wq = w_v[pl.ds(c * _CHUNK + it * _UNROLL, _LANES)]
            accs = list(accs)
            for u in range(_UNROLL):
                row = it * _UNROLL + u
                ss = [rows_v[buf, row, pl.ds(j * _LANES, _LANES)]
                      for j in range(4)]
                for j in range(4, _NVEC):
                    ss[j & 3] = ss[j & 3] + rows_v[
                        buf, row, pl.ds(j * _LANES, _LANES)]
                s = (ss[0] + ss[1]) + (ss[2] + ss[3])
                accs[u] = accs[u] + _bcast(wq, u) * s
            return tuple(accs)

        return lax.fori_loop(0, _CHUNK // _UNROLL, rowit, accs)

    accs = tuple(jnp.zeros((_LANES,), jnp.float32) for _ in range(_UNROLL))
    accs = lax.fori_loop(0, _NCHUNK, chunk_body, accs)

    parts = list(accs)
    while len(parts) > 1:
        parts = [parts[i] + parts[i + 1] for i in range(0, len(parts), 2)]
    acc_v[...] = parts[0]
    pltpu.sync_copy(acc_v, out_hbm.at[wid])


@jax.jit
def _run(ids3, cu16, table):
    mesh = plsc.VectorSubcoreMesh(core_axis_name="c", subcore_axis_name="s")
    kern = pl.kernel(
        _body,
        out_type=jax.ShapeDtypeStruct((_NW, _LANES), jnp.float32),
        mesh=mesh,
        scratch_types=[
            pltpu.VMEM((_TPW,), jnp.int32),
            pltpu.VMEM((_NBUF, _CHUNK, _D), jnp.float32),
            pltpu.VMEM((_TPW + _LANES,), jnp.float32),
            pltpu.VMEM((_LANES,), jnp.int32),
            pltpu.VMEM((_LANES,), jnp.float32),
            pltpu.SemaphoreType.DMA((_NBUF,)),
        ],
    )
    return kern(ids3, cu16, table)


def kernel(token_ids, cu_seqlens, table):
    partials = _run(token_ids.astype(jnp.int32),
                    cu_seqlens.astype(jnp.int32), table)
    return jnp.sum(partials)


# speedup vs baseline: 1.0108x; 1.0108x over previous
import jax
import jax.numpy as jnp
from jax import lax
from jax.experimental import pallas as pl
from jax.experimental.pallas import tpu as pltpu
from jax.experimental.pallas import tpu_sc as plsc

_VOCAB = 32000
_D = 512
_T = 8192
_NC = 2
_NS = 16
_NW = _NC * _NS
_TPW = _T // _NW
_CHUNK = 64
_NCHUNK = _TPW // _CHUNK
_LANES = 16
_NVEC = _D // _LANES
_UNROLL = 4


def _gat(vec, idx):
    dnums = lax.GatherDimensionNumbers(
        offset_dims=(), collapsed_slice_dims=(0,), start_index_map=(0,))
    return lax.gather(vec, idx[:, None], dnums, (1,),
                      mode=lax.GatherScatterMode.PROMISE_IN_BOUNDS)


def _bcast(vec, lane):
    return _gat(vec, jnp.full((_LANES,), lane, jnp.int32))


def _body(ids_hbm, cu_hbm, table_hbm, out_hbm, idx_v, rows_v, w_v, cu_v,
          acc_v, sems):
    cid = lax.axis_index("c")
    sid = lax.axis_index("s")
    wid = cid * _NS + sid
    base = wid * _TPW

    cu_d = pltpu.make_async_copy(cu_hbm, cu_v.at[pl.ds(0, 9)], sems.at[1])
    idx_d = pltpu.make_async_copy(
        ids_hbm.at[pl.ds(base, _TPW)], idx_v, sems.at[1])
    cu_d.start()
    idx_d.start()

    def gather_descr(c):
        return pltpu.make_async_copy(
            table_hbm.at[idx_v.at[pl.ds(c * _CHUNK, _CHUNK)]],
            rows_v.at[c & 1], sems.at[c & 1])

    idx_d.wait()
    gather_descr(0).start()
    cu_d.wait()

    lanes = lax.iota(jnp.int32, 16)
    cuv = cu_v[...]
    cks = [_bcast(cuv, k) for k in range(1, 9)]

    def wgrp(jj, carry):
        t = base + jj * _LANES + lanes
        seg = jnp.zeros((_LANES,), jnp.int32)
        for ck in cks:
            seg = seg + jnp.where(ck <= t, 1, 0)
        start = _gat(cuv, seg)
        end = _gat(cuv, seg + 1)
        pos = t - start
        seg_len = end - start
        w = ((pos + 1) * (seg_len - pos)).astype(jnp.float32)
        w_v[pl.ds(jj * _LANES, _LANES)] = w
        return carry

    lax.fori_loop(0, _TPW // _LANES, wgrp, jnp.int32(0))

    def chunk_body(c, accs):
        nxt = jnp.minimum(c + 1, _NCHUNK - 1)

        @pl.when(c + 1 < _NCHUNK)
        def _():
            gather_descr(nxt).start()

        gather_descr(c).wait()
        buf = c & 1

        def rowit(it, accs):
            wq = w_v[pl.ds(c * _CHUNK + it * _UNROLL, _LANES)]
            accs = list(accs)
            for u in range(_UNROLL):
                row = it * _UNROLL + u
                ss = [rows_v[buf, row, pl.ds(j * _LANES, _LANES)]
                      for j in range(4)]
                for j in range(4, _NVEC):
                    ss[j & 3] = ss[j & 3] + rows_v[
                        buf, row, pl.ds(j * _LANES, _LANES)]
                s = (ss[0] + ss[1]) + (ss[2] + ss[3])
                accs[u] = accs[u] + _bcast(wq, u) * s
            return tuple(accs)

        return lax.fori_loop(0, _CHUNK // _UNROLL, rowit, accs)

    accs = tuple(jnp.zeros((_LANES,), jnp.float32) for _ in range(_UNROLL))
    accs = lax.fori_loop(0, _NCHUNK, chunk_body, accs)

    parts = list(accs)
    while len(parts) > 1:
        parts = [parts[i] + parts[i + 1] for i in range(0, len(parts), 2)]
    acc_v[...] = parts[0]
    pltpu.sync_copy(acc_v, out_hbm.at[wid])


@jax.jit
def _run(ids3, cu16, table):
    mesh = plsc.VectorSubcoreMesh(core_axis_name="c", subcore_axis_name="s")
    kern = pl.kernel(
        _body,
        out_type=jax.ShapeDtypeStruct((_NW, _LANES), jnp.float32),
        mesh=mesh,
        scratch_types=[
            pltpu.VMEM((_TPW,), jnp.int32),
            pltpu.VMEM((2, _CHUNK, _D), jnp.float32),
            pltpu.VMEM((_TPW + _LANES,), jnp.float32),
            pltpu.VMEM((_LANES,), jnp.int32),
            pltpu.VMEM((_LANES,), jnp.float32),
            pltpu.SemaphoreType.DMA((2,)),
        ],
    )
    return kern(ids3, cu16, table)


def kernel(token_ids, cu_seqlens, table):
    partials = _run(token_ids.astype(jnp.int32),
                    cu_seqlens.astype(jnp.int32), table)
    return jnp.sum(partials)
